# baseline (device time: 29704 ns/iter reference)
import jax
import jax.numpy as jnp
from jax import lax
from jax.experimental import pallas as pl
from jax.experimental.pallas import tpu as pltpu

N_DEV = 8


def kernel(x, w_mat):
    m, k_loc = x.shape
    k, n = w_mat.shape
    blk = m // N_DEV

    def body(x_ref, w_ref, out_ref, xg_ref, send_sems, recv_sems):
        my = lax.axis_index("i")

        barrier_sem = pltpu.get_barrier_semaphore()
        for off in range(1, N_DEV):
            peer = lax.rem(my + off, N_DEV)
            pl.semaphore_signal(
                barrier_sem, inc=1,
                device_id=(peer,), device_id_type=pl.DeviceIdType.MESH,
            )
        pl.semaphore_wait(barrier_sem, N_DEV - 1)

        xg_ref[my] = x_ref[pl.ds(my * blk, blk), :]

        rdmas = []
        for off in range(1, N_DEV):
            peer = lax.rem(my + off, N_DEV)
            rdma = pltpu.make_async_remote_copy(
                src_ref=x_ref.at[pl.ds(peer * blk, blk), :],
                dst_ref=xg_ref.at[my],
                send_sem=send_sems.at[off],
                recv_sem=recv_sems.at[off],
                device_id=(peer,),
                device_id_type=pl.DeviceIdType.MESH,
            )
            rdma.start()
            rdmas.append(rdma)

        out_ref[...] = jnp.dot(
            xg_ref[my],
            w_ref[pl.ds(my * blk, blk), :],
            preferred_element_type=jnp.float32,
        )

        for off in range(1, N_DEV):
            rdmas[off - 1].wait_recv()
            src = lax.rem(my - off + N_DEV, N_DEV)
            out_ref[...] += jnp.dot(
                xg_ref[src],
                w_ref[pl.ds(src * blk, blk), :],
                preferred_element_type=jnp.float32,
            )

        for rdma in rdmas:
            rdma.wait_send()

    return pl.pallas_call(
        body,
        out_shape=jax.ShapeDtypeStruct((blk, n), jnp.float32),
        in_specs=[
            pl.BlockSpec(memory_space=pltpu.VMEM),
            pl.BlockSpec(memory_space=pltpu.VMEM),
        ],
        out_specs=pl.BlockSpec(memory_space=pltpu.VMEM),
        scratch_shapes=[
            pltpu.VMEM((N_DEV, blk, k_loc), jnp.float32),
            pltpu.SemaphoreType.DMA((N_DEV,)),
            pltpu.SemaphoreType.DMA((N_DEV,)),
        ],
        compiler_params=pltpu.CompilerParams(collective_id=0),
    )(x, w_mat)


# device time: 28661 ns/iter; 1.0364x vs baseline; 1.0364x over previous
import jax
import jax.numpy as jnp
from jax import lax
from jax.experimental import pallas as pl
from jax.experimental.pallas import tpu as pltpu

N_DEV = 8


def kernel(x, w_mat):
    m, k_loc = x.shape
    k, n = w_mat.shape
    blk = m // N_DEV

    def body(x_ref, w_ref, out_ref, xg_ref, w_buf, w_sems,
             send_sems, recv_sems, ready_sems):
        my = lax.axis_index("i")

        barrier_sem = pltpu.get_barrier_semaphore()
        pl.semaphore_signal(barrier_sem, inc=1)
        pl.semaphore_wait(barrier_sem, 1)

        for off in range(1, N_DEV):
            src = lax.rem(my - off + N_DEV, N_DEV)
            pl.semaphore_signal(
                ready_sems.at[off], inc=1,
                device_id=(src,), device_id_type=pl.DeviceIdType.MESH,
            )

        xg_ref[my] = x_ref[pl.ds(my * blk, blk), :]

        def w_load(j, slot):
            src = lax.rem(my - j + N_DEV, N_DEV)
            return pltpu.make_async_copy(
                w_ref.at[pl.ds(src * blk, blk), :], w_buf.at[slot],
                w_sems.at[slot],
            )

        w_load(0, 0).start()
        w_load(1, 1).start()

        sends = []
        for off in range(1, N_DEV):
            peer = lax.rem(my + off, N_DEV)
            pl.semaphore_wait(ready_sems.at[off], 1)
            rdma = pltpu.make_async_remote_copy(
                src_ref=x_ref.at[pl.ds(peer * blk, blk), :],
                dst_ref=xg_ref.at[my],
                send_sem=send_sems.at[off],
                recv_sem=recv_sems.at[off],
                device_id=(peer,),
                device_id_type=pl.DeviceIdType.MESH,
            )
            rdma.start()
            sends.append(rdma)

        out_ref[...] = jnp.zeros_like(out_ref)
        for j in range(N_DEV):
            slot = j % 2
            w_load(j, slot).wait()
            if j >= 1:
                sends[j - 1].wait_recv()
            src = lax.rem(my - j + N_DEV, N_DEV)
            out_ref[...] += jnp.dot(
                xg_ref[src], w_buf[slot], preferred_element_type=jnp.float32
            )
            if j + 2 < N_DEV:
                w_load(j + 2, slot).start()

        for s in sends:
            s.wait_send()

    return pl.pallas_call(
        body,
        out_shape=jax.ShapeDtypeStruct((blk, n), jnp.float32),
        in_specs=[
            pl.BlockSpec(memory_space=pltpu.VMEM),
            pl.BlockSpec(memory_space=pl.ANY),
        ],
        out_specs=pl.BlockSpec(memory_space=pltpu.VMEM),
        scratch_shapes=[
            pltpu.VMEM((N_DEV, blk, k_loc), jnp.float32),
            pltpu.VMEM((2, blk, n), jnp.float32),
            pltpu.SemaphoreType.DMA((2,)),
            pltpu.SemaphoreType.DMA((N_DEV,)),
            pltpu.SemaphoreType.DMA((N_DEV,)),
            pltpu.SemaphoreType.REGULAR((N_DEV,)),
        ],
        compiler_params=pltpu.CompilerParams(collective_id=0),
    )(x, w_mat)
